# Initial kernel scaffold; baseline (speedup 1.0000x reference)
import jax
import jax.numpy as jnp
from jax.experimental import pallas as pl


def kernel(position_q, position_k, embeddings_table):
    raise NotImplementedError("write your pallas kernel here")


# speedup vs baseline: 1.2542x; 1.2542x over previous
import functools

import jax
import jax.numpy as jnp
from jax import lax
from jax.experimental import pallas as pl
from jax.experimental.pallas import tpu as pltpu
from jax.experimental.pallas import tpu_sc as plsc

MAX_REL_POS = 128
NUM_UNITS = 64
NVOC = 2 * MAX_REL_POS + 1

B, LQ, LK = 8, 32, 2048
N_ROWS = B * LQ * LK
NC, NS = 2, 16
NW = NC * NS
ROWS_PER_W = N_ROWS // NW
CHUNK = 512
NCHUNK = ROWS_PER_W // CHUNK
TPC = CHUNK // 16


def _sc_body(pq_hbm, pk_hbm, table_hbm, out_hbm,
             pq_v, pk_v, table_v, rows_v, sem_out):
    wid = lax.axis_index("s") * NC + lax.axis_index("c")
    pltpu.sync_copy(pq_hbm, pq_v)
    pltpu.sync_copy(pk_hbm, pk_v)
    pltpu.sync_copy(table_hbm, table_v)
    lane = lax.iota(jnp.int32, 16)

    def chunk_body(c, carry):
        buf = c % 2
        g0 = wid * ROWS_PER_W + c * CHUNK
        bi = g0 // LK
        b = bi // LQ
        j0 = g0 % LK
        pq_b = pq_v[bi, :]
        buf16 = jnp.full((16,), buf, jnp.int32)

        @pl.when(c >= 2)
        def _():
            pltpu.make_async_copy(
                rows_v.at[0], out_hbm.at[pl.ds(0, CHUNK)], sem_out).wait()

        def t_body(t, carry2):
            kv = pk_v[pl.ds(b * LK + j0 + t * 16, 16)]
            d = kv - pq_b
            d = jnp.minimum(jnp.maximum(d, -MAX_REL_POS), MAX_REL_POS)
            idx16 = d + MAX_REL_POS
            row16 = lane + t * 16
            for f in range(NUM_UNITS):
                fv = jnp.full((16,), f, jnp.int32)
                v = plsc.load_gather(table_v, [idx16, fv])
                plsc.store_scatter(rows_v, [buf16, row16, fv], v)
            return carry2

        lax.fori_loop(0, TPC, t_body, 0)
        pltpu.make_async_copy(
            rows_v.at[buf], out_hbm.at[pl.ds(g0, CHUNK)], sem_out).start()
        return carry

    lax.fori_loop(0, NCHUNK, chunk_body, 0)
    pltpu.make_async_copy(
        rows_v.at[0], out_hbm.at[pl.ds(0, CHUNK)], sem_out).wait()
    pltpu.make_async_copy(
        rows_v.at[0], out_hbm.at[pl.ds(0, CHUNK)], sem_out).wait()


@functools.partial(
    pl.kernel,
    mesh=plsc.VectorSubcoreMesh(core_axis_name="c", subcore_axis_name="s"),
    out_type=jax.ShapeDtypeStruct((N_ROWS, NUM_UNITS), jnp.float32),
    scratch_types=[
        pltpu.VMEM((B * LQ, 16), jnp.int32),
        pltpu.VMEM((B * LK,), jnp.int32),
        pltpu.VMEM((NVOC, NUM_UNITS), jnp.float32),
        pltpu.VMEM((2, CHUNK, NUM_UNITS), jnp.float32),
        pltpu.SemaphoreType.DMA,
    ],
    compiler_params=pltpu.CompilerParams(
        use_tc_tiling_on_sc=False, needs_layout_passes=False),
)
def _sc_call(pq_hbm, pk_hbm, table_hbm, out_hbm,
             pq_v, pk_v, table_v, rows_v, sem_out):
    _sc_body(pq_hbm, pk_hbm, table_hbm, out_hbm,
             pq_v, pk_v, table_v, rows_v, sem_out)


def kernel(position_q, position_k, embeddings_table):
    pq = jnp.broadcast_to(
        position_q.reshape(-1, 1).astype(jnp.int32), (B * LQ, 16))
    pk = position_k.reshape(-1).astype(jnp.int32)
    out = _sc_call(pq, pk, embeddings_table)
    return out.reshape(B, LQ, LK, NUM_UNITS)


# speedup vs baseline: 1.6463x; 1.3127x over previous
import functools

import jax
import jax.numpy as jnp
from jax import lax
from jax.experimental import pallas as pl
from jax.experimental.pallas import tpu as pltpu
from jax.experimental.pallas import tpu_sc as plsc

MAX_REL_POS = 128
NUM_UNITS = 64
NVOC = 2 * MAX_REL_POS + 1

B, LQ, LK = 8, 32, 2048
N_ROWS = B * LQ * LK
NC, NS = 2, 16
NW = NC * NS
ROWS_PER_W = N_ROWS // NW
CHUNK = 512
NCHUNK = ROWS_PER_W // CHUNK
TPC = CHUNK // 16


def _sc_body(pq_hbm, pk_hbm, table_hbm, out_hbm,
             pq_v, pk_v, table_v, rows_v, sem_out):
    wid = lax.axis_index("s") * NC + lax.axis_index("c")
    pltpu.sync_copy(pq_hbm, pq_v)
    pltpu.sync_copy(pk_hbm, pk_v)
    pltpu.sync_copy(table_hbm, table_v)
    lane = lax.iota(jnp.int32, 16)

    def chunk_body(c, carry):
        buf = c % 2
        g0 = wid * ROWS_PER_W + c * CHUNK
        bi = g0 // LK
        b = bi // LQ
        j0 = g0 % LK
        pq_b = pq_v[bi, :]
        buf16 = jnp.full((16,), buf, jnp.int32)

        @pl.when(c >= 2)
        def _():
            pltpu.make_async_copy(
                rows_v.at[0], out_hbm.at[pl.ds(0, CHUNK)], sem_out).wait()

        def t_body(t, carry2):
            kv = pk_v[pl.ds(b * LK + j0 + t * 16, 16)]
            d = kv - pq_b
            d = jnp.minimum(jnp.maximum(d, -MAX_REL_POS), MAX_REL_POS)
            idx16 = d + MAX_REL_POS
            row16 = lane + t * 16
            G = 16
            for f0 in range(0, NUM_UNITS, G):
                fvs = [jnp.full((16,), f0 + u, jnp.int32) for u in range(G)]
                vs = [plsc.load_gather(table_v, [idx16, fvs[u]])
                      for u in range(G)]
                for u in range(G):
                    plsc.store_scatter(rows_v, [buf16, row16, fvs[u]], vs[u])
            return carry2

        lax.fori_loop(0, TPC, t_body, 0)
        pltpu.make_async_copy(
            rows_v.at[buf], out_hbm.at[pl.ds(g0, CHUNK)], sem_out).start()
        return carry

    lax.fori_loop(0, NCHUNK, chunk_body, 0)
    pltpu.make_async_copy(
        rows_v.at[0], out_hbm.at[pl.ds(0, CHUNK)], sem_out).wait()
    pltpu.make_async_copy(
        rows_v.at[0], out_hbm.at[pl.ds(0, CHUNK)], sem_out).wait()


@functools.partial(
    pl.kernel,
    mesh=plsc.VectorSubcoreMesh(core_axis_name="c", subcore_axis_name="s"),
    out_type=jax.ShapeDtypeStruct((N_ROWS, NUM_UNITS), jnp.float32),
    scratch_types=[
        pltpu.VMEM((B * LQ, 16), jnp.int32),
        pltpu.VMEM((B * LK,), jnp.int32),
        pltpu.VMEM((NVOC, NUM_UNITS), jnp.float32),
        pltpu.VMEM((2, CHUNK, NUM_UNITS), jnp.float32),
        pltpu.SemaphoreType.DMA,
    ],
    compiler_params=pltpu.CompilerParams(
        use_tc_tiling_on_sc=False, needs_layout_passes=False),
)
def _sc_call(pq_hbm, pk_hbm, table_hbm, out_hbm,
             pq_v, pk_v, table_v, rows_v, sem_out):
    _sc_body(pq_hbm, pk_hbm, table_hbm, out_hbm,
             pq_v, pk_v, table_v, rows_v, sem_out)


def kernel(position_q, position_k, embeddings_table):
    pq = jnp.broadcast_to(
        position_q.reshape(-1, 1).astype(jnp.int32), (B * LQ, 16))
    pk = position_k.reshape(-1).astype(jnp.int32)
    out = _sc_call(pq, pk, embeddings_table)
    return out.reshape(B, LQ, LK, NUM_UNITS)


# speedup vs baseline: 4.7363x; 2.8769x over previous
import functools

import jax
import jax.numpy as jnp
from jax import lax
from jax.experimental import pallas as pl
from jax.experimental.pallas import tpu as pltpu
from jax.experimental.pallas import tpu_sc as plsc

MAX_REL_POS = 128
NUM_UNITS = 64
NVOC = 2 * MAX_REL_POS + 1

B, LQ, LK = 8, 32, 2048
N_ROWS = B * LQ * LK
NC, NS = 2, 16
NW = NC * NS
ROWS_PER_W = N_ROWS // NW
CHUNK = 512
NCHUNK = ROWS_PER_W // CHUNK
TPC = CHUNK // 16


def _sc_body(pq_hbm, pk_hbm, table_hbm, out_hbm,
             pq_v, pk_v, table_v, rows_v, sem_out):
    wid = lax.axis_index("s") * NC + lax.axis_index("c")
    pltpu.sync_copy(pq_hbm, pq_v)
    pltpu.sync_copy(pk_hbm, pk_v)
    pltpu.sync_copy(table_hbm, table_v)
    lane = lax.iota(jnp.int32, 16)

    def chunk_body(c, carry):
        buf = c % 2
        g0 = wid * ROWS_PER_W + c * CHUNK
        bi = g0 // LK
        b = bi // LQ
        j0 = g0 % LK
        pq_b = pq_v[bi, :]

        @pl.when(c >= 2)
        def _():
            pltpu.make_async_copy(
                rows_v.at[0], out_hbm.at[pl.ds(0, CHUNK)], sem_out).wait()

        def t_body(t, carry2):
            kv = pk_v[pl.ds(b * LK + j0 + t * 16, 16)]
            d = kv - pq_b
            d = jnp.minimum(jnp.maximum(d, -MAX_REL_POS), MAX_REL_POS)
            a16 = (d + MAX_REL_POS) * NUM_UNITS
            for r in range(0, 16, 2):
                base0 = jnp.full((16,), a16[r], jnp.int32)
                base1 = jnp.full((16,), a16[r + 1], jnp.int32)
                vs = [plsc.load_gather(table_v, [base0 + (lane + f0)])
                      for f0 in range(0, NUM_UNITS, 16)]
                vs += [plsc.load_gather(table_v, [base1 + (lane + f0)])
                       for f0 in range(0, NUM_UNITS, 16)]
                for u, f0 in enumerate(range(0, NUM_UNITS, 16)):
                    rows_v[buf, t * 16 + r, pl.ds(f0, 16)] = vs[u]
                for u, f0 in enumerate(range(0, NUM_UNITS, 16)):
                    rows_v[buf, t * 16 + r + 1, pl.ds(f0, 16)] = vs[4 + u]
            return carry2

        lax.fori_loop(0, TPC, t_body, 0)
        pltpu.make_async_copy(
            rows_v.at[buf], out_hbm.at[pl.ds(g0, CHUNK)], sem_out).start()
        return carry

    lax.fori_loop(0, NCHUNK, chunk_body, 0)
    pltpu.make_async_copy(
        rows_v.at[0], out_hbm.at[pl.ds(0, CHUNK)], sem_out).wait()
    pltpu.make_async_copy(
        rows_v.at[0], out_hbm.at[pl.ds(0, CHUNK)], sem_out).wait()


@functools.partial(
    pl.kernel,
    mesh=plsc.VectorSubcoreMesh(core_axis_name="c", subcore_axis_name="s"),
    out_type=jax.ShapeDtypeStruct((N_ROWS, NUM_UNITS), jnp.float32),
    scratch_types=[
        pltpu.VMEM((B * LQ, 16), jnp.int32),
        pltpu.VMEM((B * LK,), jnp.int32),
        pltpu.VMEM((NVOC * NUM_UNITS,), jnp.float32),
        pltpu.VMEM((2, CHUNK, NUM_UNITS), jnp.float32),
        pltpu.SemaphoreType.DMA,
    ],
    compiler_params=pltpu.CompilerParams(
        use_tc_tiling_on_sc=False, needs_layout_passes=False),
)
def _sc_call(pq_hbm, pk_hbm, table_hbm, out_hbm,
             pq_v, pk_v, table_v, rows_v, sem_out):
    _sc_body(pq_hbm, pk_hbm, table_hbm, out_hbm,
             pq_v, pk_v, table_v, rows_v, sem_out)


def kernel(position_q, position_k, embeddings_table):
    pq = jnp.broadcast_to(
        position_q.reshape(-1, 1).astype(jnp.int32), (B * LQ, 16))
    pk = position_k.reshape(-1).astype(jnp.int32)
    tab = embeddings_table.reshape(-1)
    out = _sc_call(pq, pk, tab)
    return out.reshape(B, LQ, LK, NUM_UNITS)
